# Initial kernel scaffold; baseline (speedup 1.0000x reference)
import jax
import jax.numpy as jnp
from jax.experimental import pallas as pl


def kernel(node_feats, segment_ids):
    raise NotImplementedError("write your pallas kernel here")


# speedup vs baseline: 2.0169x; 2.0169x over previous
import functools

import jax
import jax.numpy as jnp
from jax import lax
from jax.experimental import pallas as pl
from jax.experimental.pallas import tpu as pltpu
from jax.experimental.pallas import tpu_sc as plsc

N_ROWS = 100000
D = 256
DC = D + 16
NSEG = 64
CHUNK = 160
NCHUNK = N_ROWS // CHUNK
NW = 32
CHUNKS_PER_W = -(-NCHUNK // NW)
NT = 16
ROWS_PER_TILE = NSEG // NT

_mesh = plsc.VectorSubcoreMesh(core_axis_name="c", subcore_axis_name="s")


@functools.partial(
    pl.kernel,
    mesh=_mesh,
    compiler_params=pltpu.CompilerParams(needs_layout_passes=False),
    out_type=jax.ShapeDtypeStruct((2, NSEG, DC), jnp.float32),
    scratch_types=[
        pltpu.VMEM((CHUNK, D), jnp.float32),
        pltpu.VMEM((CHUNK,), jnp.int32),
        pltpu.VMEM((NSEG, DC), jnp.float32),
        pltpu.VMEM((NT, ROWS_PER_TILE, DC), jnp.float32),
        pltpu.VMEM_SHARED((NT, NSEG, DC), jnp.float32),
    ],
)
def _seg_mean_sc(feats_hbm, seg_hbm, zero_hbm, out_hbm,
                 buf, segbuf, acc, slices, shacc):
    c = lax.axis_index("c")
    s = lax.axis_index("s")
    wid = s * 2 + c

    pltpu.sync_copy(zero_hbm, acc)

    iota16 = lax.iota(jnp.int32, 16)
    ones16 = jnp.full((16,), 1.0, jnp.float32)

    def do_chunk(cid):
        base = pl.multiple_of(cid * CHUNK, 8)
        pltpu.sync_copy(feats_hbm.at[pl.ds(base, CHUNK), :], buf)
        pltpu.sync_copy(seg_hbm.at[pl.ds(base, CHUNK)], segbuf)

        def row(i, carry):
            sid = plsc.load_gather(segbuf, [jnp.full((16,), 0, jnp.int32) + i])
            plsc.addupdate_scatter(acc, [sid, iota16 + D], ones16)
            for j in range(D // 16):
                v = buf[i, pl.ds(j * 16, 16)]
                plsc.addupdate_scatter(acc, [sid, iota16 + (j * 16)], v)
            return carry

        lax.fori_loop(0, CHUNK, row, 0)

    for k in range(CHUNKS_PER_W):
        cid = wid + NW * k

        @pl.when(cid < NCHUNK)
        def _chunk():
            do_chunk(cid)

    pltpu.sync_copy(acc, shacc.at[s])
    plsc.subcore_barrier()

    r0 = s * ROWS_PER_TILE
    pltpu.sync_copy(shacc.at[:, pl.ds(r0, ROWS_PER_TILE), :], slices)
    for r in range(ROWS_PER_TILE):
        for j in range(DC // 16):
            tot = slices[0, r, pl.ds(j * 16, 16)]
            for t in range(1, NT):
                tot = tot + slices[t, r, pl.ds(j * 16, 16)]
            acc[r, pl.ds(j * 16, 16)] = tot
    pltpu.sync_copy(acc.at[pl.ds(0, ROWS_PER_TILE), :],
                    out_hbm.at[c, pl.ds(r0, ROWS_PER_TILE), :])


def kernel(node_feats, segment_ids):
    zero = jnp.zeros((NSEG, DC), jnp.float32)
    partials = _seg_mean_sc(node_feats, segment_ids, zero)
    p = partials[0] + partials[1]
    sums = p[:, :D]
    counts = jnp.clip(p[:, D], 1.0)
    return sums / counts[:, None]


# speedup vs baseline: 2.5392x; 1.2590x over previous
import functools

import jax
import jax.numpy as jnp
from jax import lax
from jax.experimental import pallas as pl
from jax.experimental.pallas import tpu as pltpu
from jax.experimental.pallas import tpu_sc as plsc

N_ROWS = 100000
D = 256
DC = D + 16
NSEG = 64
CHUNK = 80
NCHUNK = N_ROWS // CHUNK
NW = 32
CHUNKS_PER_W = -(-NCHUNK // NW)
NT = 16
ROWS_PER_TILE = NSEG // NT

_mesh = plsc.VectorSubcoreMesh(core_axis_name="c", subcore_axis_name="s")


@functools.partial(
    pl.kernel,
    mesh=_mesh,
    compiler_params=pltpu.CompilerParams(needs_layout_passes=False),
    out_type=jax.ShapeDtypeStruct((2, NSEG, DC), jnp.float32),
    scratch_types=[
        pltpu.VMEM((2, CHUNK, D), jnp.float32),
        pltpu.VMEM((CHUNK,), jnp.int32),
        pltpu.VMEM((CHUNK,), jnp.int32),
        pltpu.VMEM((NSEG, D), jnp.float32),
        pltpu.VMEM((NSEG * 16,), jnp.float32),
        pltpu.VMEM((NT, ROWS_PER_TILE, D), jnp.float32),
        pltpu.VMEM((NT, NSEG * 16), jnp.float32),
        pltpu.VMEM((ROWS_PER_TILE, DC), jnp.float32),
        pltpu.VMEM_SHARED((NT, NSEG, D), jnp.float32),
        pltpu.VMEM_SHARED((NT, NSEG * 16), jnp.float32),
        pltpu.SemaphoreType.DMA,
        pltpu.SemaphoreType.DMA,
    ],
)
def _seg_mean_sc(feats_hbm, seg_hbm, zero_hbm, zeroc_hbm, out_hbm,
                 buf, segbuf0, segbuf1, facc, cacc, fsl, csl, outbuf,
                 shf, shc, sem0, sem1):
    c = lax.axis_index("c")
    s = lax.axis_index("s")
    wid = s * 2 + c
    sems = [sem0, sem1]
    segbufs = [segbuf0, segbuf1]

    pltpu.sync_copy(zero_hbm, facc)
    pltpu.sync_copy(zeroc_hbm, cacc)

    iota16 = lax.iota(jnp.int32, 16)
    ones16 = jnp.full((16,), 1.0, jnp.float32)

    def copies(k, b):
        cid = wid + NW * k
        base = pl.multiple_of(cid * CHUNK, 8)
        return (
            (feats_hbm.at[pl.ds(base, CHUNK), :], buf.at[b], sems[b]),
            (seg_hbm.at[pl.ds(base, CHUNK)], segbufs[b], sems[b]),
        )

    def start(k, b):
        for args in copies(k, b):
            pltpu.async_copy(*args)

    def wait(k, b):
        for args in copies(k, b):
            pltpu.make_async_copy(*args).wait()

    def process(b):
        def row(i, carry):
            sid = plsc.load_gather(segbufs[b],
                                   [jnp.full((16,), 0, jnp.int32) + i])
            plsc.addupdate_scatter(cacc, [sid * 16 + iota16], ones16)
            for j in range(D // 16):
                v = buf[b, i, pl.ds(j * 16, 16)]
                plsc.addupdate_scatter(facc, [sid, iota16 + (j * 16)], v)
            return carry

        lax.fori_loop(0, CHUNK, row, 0, unroll=2)

    start(0, 0)
    start(1, 1)

    def ring_round(i2, carry):
        for b in range(2):
            k = i2 * 2 + b

            @pl.when(wid + NW * k < NCHUNK)
            def _slot():
                wait(k, b)
                process(b)

                @pl.when(wid + NW * (k + 2) < NCHUNK)
                def _refill():
                    start(k + 2, b)
        return carry

    lax.fori_loop(0, CHUNKS_PER_W // 2, ring_round, 0)

    pltpu.sync_copy(facc, shf.at[s])
    pltpu.sync_copy(cacc, shc.at[s])
    plsc.subcore_barrier()

    r0 = s * ROWS_PER_TILE
    pltpu.sync_copy(shf.at[:, pl.ds(r0, ROWS_PER_TILE), :], fsl)
    pltpu.sync_copy(shc, csl)
    for r in range(ROWS_PER_TILE):
        for j in range(D // 16):
            tot = fsl[0, r, pl.ds(j * 16, 16)]
            for t in range(1, NT):
                tot = tot + fsl[t, r, pl.ds(j * 16, 16)]
            outbuf[r, pl.ds(j * 16, 16)] = tot
        cnt = csl[0, pl.ds(r0 * 16 + r * 16, 16)]
        for t in range(1, NT):
            cnt = cnt + csl[t, pl.ds(r0 * 16 + r * 16, 16)]
        outbuf[r, pl.ds(D, 16)] = cnt
    pltpu.sync_copy(outbuf, out_hbm.at[c, pl.ds(r0, ROWS_PER_TILE), :])


def kernel(node_feats, segment_ids):
    zero = jnp.zeros((NSEG, D), jnp.float32)
    zeroc = jnp.zeros((NSEG * 16,), jnp.float32)
    partials = _seg_mean_sc(node_feats, segment_ids, zero, zeroc)
    p = partials[0] + partials[1]
    sums = p[:, :D]
    counts = jnp.clip(p[:, D], 1.0)
    return sums / counts[:, None]


# speedup vs baseline: 5.2202x; 2.0559x over previous
import functools

import jax
import jax.numpy as jnp
from jax import lax
from jax.experimental import pallas as pl
from jax.experimental.pallas import tpu as pltpu
from jax.experimental.pallas import tpu_sc as plsc

N_ROWS = 100000
D = 256
DC = D + 16
NSEG = 64
CHUNK = 80
NCHUNK = N_ROWS // CHUNK
NW = 32
CHUNKS_PER_W = -(-NCHUNK // NW)
NT = 16
ROWS_PER_TILE = NSEG // NT

_mesh = plsc.VectorSubcoreMesh(core_axis_name="c", subcore_axis_name="s")


@functools.partial(
    pl.kernel,
    mesh=_mesh,
    compiler_params=pltpu.CompilerParams(needs_layout_passes=False),
    out_type=jax.ShapeDtypeStruct((2, NSEG, DC), jnp.float32),
    scratch_types=[
        pltpu.VMEM((2, CHUNK, D), jnp.float32),
        pltpu.VMEM((CHUNK,), jnp.int32),
        pltpu.VMEM((CHUNK,), jnp.int32),
        pltpu.VMEM((NSEG, D), jnp.float32),
        pltpu.VMEM((NSEG * 16,), jnp.float32),
        pltpu.VMEM((NT, ROWS_PER_TILE, D), jnp.float32),
        pltpu.VMEM((NT, NSEG * 16), jnp.float32),
        pltpu.VMEM((ROWS_PER_TILE, DC), jnp.float32),
        pltpu.VMEM_SHARED((NT, NSEG, D), jnp.float32),
        pltpu.VMEM_SHARED((NT, NSEG * 16), jnp.float32),
        pltpu.SemaphoreType.DMA,
        pltpu.SemaphoreType.DMA,
    ],
)
def _seg_mean_sc(feats_hbm, seg_hbm, zero_hbm, zeroc_hbm, out_hbm,
                 buf, segbuf0, segbuf1, facc, cacc, fsl, csl, outbuf,
                 shf, shc, sem0, sem1):
    c = lax.axis_index("c")
    s = lax.axis_index("s")
    wid = s * 2 + c
    sems = [sem0, sem1]
    segbufs = [segbuf0, segbuf1]

    pltpu.sync_copy(zero_hbm, facc)
    pltpu.sync_copy(zeroc_hbm, cacc)

    iota16 = lax.iota(jnp.int32, 16)
    ones16 = jnp.full((16,), 1.0, jnp.float32)

    def copies(k, b):
        cid = wid + NW * k
        base = pl.multiple_of(cid * CHUNK, 8)
        return (
            (feats_hbm.at[pl.ds(base, CHUNK), :], buf.at[b], sems[b]),
            (seg_hbm.at[pl.ds(base, CHUNK)], segbufs[b], sems[b]),
        )

    def start(k, b):
        for args in copies(k, b):
            pltpu.async_copy(*args)

    def wait(k, b):
        for args in copies(k, b):
            pltpu.make_async_copy(*args).wait()

    def scatter_row(b, i):
        sid = plsc.load_gather(segbufs[b],
                               [jnp.full((16,), 0, jnp.int32) + i])
        plsc.addupdate_scatter(cacc, [sid * 16 + iota16], ones16)
        for j in range(D // 16):
            v = buf[b, i, pl.ds(j * 16, 16)]
            plsc.addupdate_scatter(facc, [sid, iota16 + (j * 16)], v)

    def process(b):
        def group(g, carry):
            base = g * 16
            svec = segbufs[b][pl.ds(base, 16)]
            all_eq = jnp.all(svec == lax.rev(svec, (0,)))

            @pl.when(all_eq)
            def _fast():
                plsc.addupdate_scatter(cacc, [svec * 16 + iota16],
                                       jnp.full((16,), 16.0, jnp.float32))
                for j in range(D // 16):
                    tot = buf[b, base, pl.ds(j * 16, 16)]
                    for r in range(1, 16):
                        tot = tot + buf[b, base + r, pl.ds(j * 16, 16)]
                    plsc.addupdate_scatter(facc, [svec, iota16 + (j * 16)],
                                           tot)

            @pl.when(jnp.logical_not(all_eq))
            def _slow():
                def rowfn(i, cc):
                    scatter_row(b, i)
                    return cc

                lax.fori_loop(base, base + 16, rowfn, 0)

            return carry

        lax.fori_loop(0, CHUNK // 16, group, 0)

    start(0, 0)
    start(1, 1)

    def ring_round(i2, carry):
        for b in range(2):
            k = i2 * 2 + b

            @pl.when(wid + NW * k < NCHUNK)
            def _slot():
                wait(k, b)
                process(b)

                @pl.when(wid + NW * (k + 2) < NCHUNK)
                def _refill():
                    start(k + 2, b)
        return carry

    lax.fori_loop(0, CHUNKS_PER_W // 2, ring_round, 0)

    pltpu.sync_copy(facc, shf.at[s])
    pltpu.sync_copy(cacc, shc.at[s])
    plsc.subcore_barrier()

    r0 = s * ROWS_PER_TILE
    pltpu.sync_copy(shf.at[:, pl.ds(r0, ROWS_PER_TILE), :], fsl)
    pltpu.sync_copy(shc, csl)
    for r in range(ROWS_PER_TILE):
        for j in range(D // 16):
            tot = fsl[0, r, pl.ds(j * 16, 16)]
            for t in range(1, NT):
                tot = tot + fsl[t, r, pl.ds(j * 16, 16)]
            outbuf[r, pl.ds(j * 16, 16)] = tot
        cnt = csl[0, pl.ds(r0 * 16 + r * 16, 16)]
        for t in range(1, NT):
            cnt = cnt + csl[t, pl.ds(r0 * 16 + r * 16, 16)]
        outbuf[r, pl.ds(D, 16)] = cnt
    pltpu.sync_copy(outbuf, out_hbm.at[c, pl.ds(r0, ROWS_PER_TILE), :])


def kernel(node_feats, segment_ids):
    zero = jnp.zeros((NSEG, D), jnp.float32)
    zeroc = jnp.zeros((NSEG * 16,), jnp.float32)
    partials = _seg_mean_sc(node_feats, segment_ids, zero, zeroc)
    p = partials[0] + partials[1]
    sums = p[:, :D]
    counts = jnp.clip(p[:, D], 1.0)
    return sums / counts[:, None]


# speedup vs baseline: 5.5151x; 1.0565x over previous
import functools

import jax
import jax.numpy as jnp
from jax import lax
from jax.experimental import pallas as pl
from jax.experimental.pallas import tpu as pltpu
from jax.experimental.pallas import tpu_sc as plsc

N_ROWS = 100000
D = 256
DC = D + 16
NSEG = 64
CHUNK = 80
NCHUNK = N_ROWS // CHUNK
NW = 32
CHUNKS_PER_W = -(-NCHUNK // NW)
NT = 16
ROWS_PER_TILE = NSEG // NT

_mesh = plsc.VectorSubcoreMesh(core_axis_name="c", subcore_axis_name="s")


@functools.partial(
    pl.kernel,
    mesh=_mesh,
    compiler_params=pltpu.CompilerParams(needs_layout_passes=False),
    out_type=jax.ShapeDtypeStruct((2, NSEG, DC), jnp.float32),
    scratch_types=[
        pltpu.VMEM((2, CHUNK, D), jnp.float32),
        pltpu.VMEM((CHUNK,), jnp.int32),
        pltpu.VMEM((CHUNK,), jnp.int32),
        pltpu.VMEM((NSEG, D), jnp.float32),
        pltpu.VMEM((NSEG * 16,), jnp.float32),
        pltpu.VMEM((NT, ROWS_PER_TILE, D), jnp.float32),
        pltpu.VMEM((NT, NSEG * 16), jnp.float32),
        pltpu.VMEM((ROWS_PER_TILE, DC), jnp.float32),
        pltpu.VMEM_SHARED((NT, NSEG, D), jnp.float32),
        pltpu.VMEM_SHARED((NT, NSEG * 16), jnp.float32),
        pltpu.SemaphoreType.DMA,
        pltpu.SemaphoreType.DMA,
    ],
)
def _seg_mean_sc(feats_hbm, seg_hbm, zero_hbm, zeroc_hbm, out_hbm,
                 buf, segbuf0, segbuf1, facc, cacc, fsl, csl, outbuf,
                 shf, shc, sem0, sem1):
    c = lax.axis_index("c")
    s = lax.axis_index("s")
    wid = s * 2 + c
    sems = [sem0, sem1]
    segbufs = [segbuf0, segbuf1]

    pltpu.sync_copy(zero_hbm, facc)
    pltpu.sync_copy(zeroc_hbm, cacc)

    iota16 = lax.iota(jnp.int32, 16)
    ones16 = jnp.full((16,), 1.0, jnp.float32)

    def copies(k, b):
        cid = wid + NW * k
        base = pl.multiple_of(cid * CHUNK, 8)
        return (
            (feats_hbm.at[pl.ds(base, CHUNK), :], buf.at[b], sems[b]),
            (seg_hbm.at[pl.ds(base, CHUNK)], segbufs[b], sems[b]),
        )

    def start(k, b):
        for args in copies(k, b):
            pltpu.async_copy(*args)

    def wait(k, b):
        for args in copies(k, b):
            pltpu.make_async_copy(*args).wait()

    def scatter_row(b, i):
        sid = plsc.load_gather(segbufs[b],
                               [jnp.full((16,), 0, jnp.int32) + i])
        plsc.addupdate_scatter(cacc, [sid * 16 + iota16], ones16)
        for j in range(D // 16):
            v = buf[b, i, pl.ds(j * 16, 16)]
            plsc.addupdate_scatter(facc, [sid, iota16 + (j * 16)], v)

    zeros16 = jnp.zeros((16,), jnp.float32)

    def flush(cur, cnt, regs):
        plsc.addupdate_scatter(cacc, [cur * 16 + iota16], cnt)
        for j in range(D // 16):
            plsc.addupdate_scatter(facc, [cur, iota16 + (j * 16)], regs[j])

    def process(b):
        def group(g, carry):
            cur, cnt = carry[0], carry[1]
            regs = carry[2:]
            base = g * 16
            svec = segbufs[b][pl.ds(base, 16)]
            uni = jnp.all(svec == lax.rev(svec, (0,)))

            def uniform_case():
                tots = []
                for j in range(D // 16):
                    vals = [buf[b, base + r, pl.ds(j * 16, 16)]
                            for r in range(16)]
                    while len(vals) > 1:
                        vals = [vals[i] + vals[i + 1]
                                for i in range(0, len(vals), 2)]
                    tots.append(vals[0])

                def same_fn():
                    return (cur, cnt + 16.0,
                            *[r + t for r, t in zip(regs, tots)])

                def diff_fn():
                    flush(cur, cnt, regs)
                    return (svec, jnp.full((16,), 16.0, jnp.float32), *tots)

                return lax.cond(jnp.all(svec == cur), same_fn, diff_fn)

            def slow_case():
                flush(cur, cnt, regs)

                def rowfn(i, cc):
                    scatter_row(b, i)
                    return cc

                lax.fori_loop(base, base + 16, rowfn, 0)
                return (svec, zeros16, *([zeros16] * (D // 16)))

            return lax.cond(uni, uniform_case, slow_case)

        init = (jnp.zeros((16,), jnp.int32), zeros16,
                *([zeros16] * (D // 16)))
        final = lax.fori_loop(0, CHUNK // 16, group, init)
        flush(final[0], final[1], final[2:])

    start(0, 0)
    start(1, 1)

    def ring_round(i2, carry):
        for b in range(2):
            k = i2 * 2 + b

            @pl.when(wid + NW * k < NCHUNK)
            def _slot():
                wait(k, b)
                process(b)

                @pl.when(wid + NW * (k + 2) < NCHUNK)
                def _refill():
                    start(k + 2, b)
        return carry

    lax.fori_loop(0, CHUNKS_PER_W // 2, ring_round, 0)

    pltpu.sync_copy(facc, shf.at[s])
    pltpu.sync_copy(cacc, shc.at[s])
    plsc.subcore_barrier()

    r0 = s * ROWS_PER_TILE
    pltpu.sync_copy(shf.at[:, pl.ds(r0, ROWS_PER_TILE), :], fsl)
    pltpu.sync_copy(shc, csl)
    for r in range(ROWS_PER_TILE):
        for j in range(D // 16):
            tot = fsl[0, r, pl.ds(j * 16, 16)]
            for t in range(1, NT):
                tot = tot + fsl[t, r, pl.ds(j * 16, 16)]
            outbuf[r, pl.ds(j * 16, 16)] = tot
        cnt = csl[0, pl.ds(r0 * 16 + r * 16, 16)]
        for t in range(1, NT):
            cnt = cnt + csl[t, pl.ds(r0 * 16 + r * 16, 16)]
        outbuf[r, pl.ds(D, 16)] = cnt
    pltpu.sync_copy(outbuf, out_hbm.at[c, pl.ds(r0, ROWS_PER_TILE), :])


def kernel(node_feats, segment_ids):
    zero = jnp.zeros((NSEG, D), jnp.float32)
    zeroc = jnp.zeros((NSEG * 16,), jnp.float32)
    partials = _seg_mean_sc(node_feats, segment_ids, zero, zeroc)
    p = partials[0] + partials[1]
    sums = p[:, :D]
    counts = jnp.clip(p[:, D], 1.0)
    return sums / counts[:, None]
